# baseline (speedup 1.0000x reference)
import jax
import jax.numpy as jnp
from jax.experimental import pallas as pl
from jax.experimental.pallas import tpu as pltpu

_ROWS, _COLS = 16384, 1024
_NCHUNK = 8
_CHUNK = _ROWS // _NCHUNK
_PATCH = 8


def _dma_copy_patch(x_hbm, y_hbm, o_hbm, xv, yv, copy_sems, xsem, ysem, wsem):
    copies = []
    for i in range(_NCHUNK):
        c = pltpu.make_async_copy(
            x_hbm.at[pl.ds(i * _CHUNK, _CHUNK)],
            o_hbm.at[pl.ds(i * _CHUNK, _CHUNK)],
            copy_sems.at[i],
        )
        c.start()
        copies.append(c)
    cx = pltpu.make_async_copy(x_hbm.at[pl.ds(0, _PATCH)], xv, xsem)
    cx.start()
    cy = pltpu.make_async_copy(y_hbm.at[pl.ds(0, _PATCH)], yv, ysem)
    cy.start()
    cx.wait()
    cy.wait()
    r = jax.lax.broadcasted_iota(jnp.int32, (_PATCH, _COLS), 0)
    c = jax.lax.broadcasted_iota(jnp.int32, (_PATCH, _COLS), 1)
    xv[...] = jnp.where((r == 0) & (c == 3), yv[0, 2], xv[...])
    copies[0].wait()
    wb = pltpu.make_async_copy(xv, o_hbm.at[pl.ds(0, _PATCH)], wsem)
    wb.start()
    wb.wait()
    for c_ in copies[1:]:
        c_.wait()


def kernel(x, y):
    return pl.pallas_call(
        _dma_copy_patch,
        in_specs=[
            pl.BlockSpec(memory_space=pl.ANY),
            pl.BlockSpec(memory_space=pl.ANY),
        ],
        out_specs=pl.BlockSpec(memory_space=pl.ANY),
        out_shape=jax.ShapeDtypeStruct((_ROWS, _COLS), x.dtype),
        scratch_shapes=[
            pltpu.VMEM((_PATCH, _COLS), jnp.float32),
            pltpu.VMEM((_PATCH, _COLS), jnp.float32),
            pltpu.SemaphoreType.DMA((_NCHUNK,)),
            pltpu.SemaphoreType.DMA,
            pltpu.SemaphoreType.DMA,
            pltpu.SemaphoreType.DMA,
        ],
    )(x, y)


# speedup vs baseline: 43.1184x; 43.1184x over previous
import jax
import jax.numpy as jnp
from jax.experimental import pallas as pl
from jax.experimental.pallas import tpu as pltpu

_ROWS, _COLS = 16384, 1024
_BLK = 512


def _copy_patch(x_ref, y_ref, o_ref):
    i = pl.program_id(0)

    @pl.when(i > 0)
    def _plain():
        o_ref[...] = x_ref[...]

    @pl.when(i == 0)
    def _patched():
        blk = x_ref[...]
        r = jax.lax.broadcasted_iota(jnp.int32, blk.shape, 0)
        c = jax.lax.broadcasted_iota(jnp.int32, blk.shape, 1)
        o_ref[...] = jnp.where((r == 0) & (c == 3), y_ref[0, 2], blk)


def kernel(x, y):
    return pl.pallas_call(
        _copy_patch,
        grid=(_ROWS // _BLK,),
        in_specs=[
            pl.BlockSpec((_BLK, _COLS), lambda i: (i, 0)),
            pl.BlockSpec((8, _COLS), lambda i: (0, 0)),
        ],
        out_specs=pl.BlockSpec((_BLK, _COLS), lambda i: (i, 0)),
        out_shape=jax.ShapeDtypeStruct((_ROWS, _COLS), x.dtype),
        compiler_params=pltpu.CompilerParams(
            dimension_semantics=("parallel",),
        ),
    )(x, y)


# speedup vs baseline: 47.0418x; 1.0910x over previous
import jax
import jax.numpy as jnp
from jax.experimental import pallas as pl
from jax.experimental.pallas import tpu as pltpu

_ROWS, _COLS = 16384, 1024
_BLK = 1024


def _copy_patch(x_ref, y_ref, o_ref):
    i = pl.program_id(0)

    @pl.when(i > 0)
    def _plain():
        o_ref[...] = x_ref[...]

    @pl.when(i == 0)
    def _patched():
        blk = x_ref[...]
        r = jax.lax.broadcasted_iota(jnp.int32, blk.shape, 0)
        c = jax.lax.broadcasted_iota(jnp.int32, blk.shape, 1)
        o_ref[...] = jnp.where((r == 0) & (c == 3), y_ref[0, 2], blk)


def kernel(x, y):
    return pl.pallas_call(
        _copy_patch,
        grid=(_ROWS // _BLK,),
        in_specs=[
            pl.BlockSpec((_BLK, _COLS), lambda i: (i, 0)),
            pl.BlockSpec((8, _COLS), lambda i: (0, 0)),
        ],
        out_specs=pl.BlockSpec((_BLK, _COLS), lambda i: (i, 0)),
        out_shape=jax.ShapeDtypeStruct((_ROWS, _COLS), x.dtype),
        compiler_params=pltpu.CompilerParams(
            dimension_semantics=("parallel",),
        ),
    )(x, y)


# speedup vs baseline: 48.7989x; 1.0374x over previous
import jax
import jax.numpy as jnp
from jax.experimental import pallas as pl
from jax.experimental.pallas import tpu as pltpu

_ROWS, _COLS = 16384, 1024
_BLK = 2048


def _copy_patch(x_ref, y_ref, o_ref):
    i = pl.program_id(0)

    @pl.when(i > 0)
    def _plain():
        o_ref[...] = x_ref[...]

    @pl.when(i == 0)
    def _patched():
        blk = x_ref[...]
        r = jax.lax.broadcasted_iota(jnp.int32, blk.shape, 0)
        c = jax.lax.broadcasted_iota(jnp.int32, blk.shape, 1)
        o_ref[...] = jnp.where((r == 0) & (c == 3), y_ref[0, 2], blk)


def kernel(x, y):
    return pl.pallas_call(
        _copy_patch,
        grid=(_ROWS // _BLK,),
        in_specs=[
            pl.BlockSpec((_BLK, _COLS), lambda i: (i, 0)),
            pl.BlockSpec((8, _COLS), lambda i: (0, 0)),
        ],
        out_specs=pl.BlockSpec((_BLK, _COLS), lambda i: (i, 0)),
        out_shape=jax.ShapeDtypeStruct((_ROWS, _COLS), x.dtype),
        compiler_params=pltpu.CompilerParams(
            dimension_semantics=("parallel",),
        ),
    )(x, y)


# speedup vs baseline: 49.4704x; 1.0138x over previous
import jax
import jax.numpy as jnp
from jax.experimental import pallas as pl
from jax.experimental.pallas import tpu as pltpu

_ROWS, _COLS = 16384, 1024
_CB = 1024
_NC = _ROWS // _CB
_K = 8
_L = 4


def _dma_pipeline(x_hbm, y_hbm, o_hbm, buf, yv, in_sems, out_sems, ysem):
    cy = pltpu.make_async_copy(y_hbm.at[pl.ds(0, 8)], yv, ysem)
    cy.start()

    ins = [None] * _NC
    outs = [None] * _NC

    def start_in(c):
        ins[c] = pltpu.make_async_copy(
            x_hbm.at[pl.ds(c * _CB, _CB)],
            buf.at[pl.ds((c % _K) * _CB, _CB)],
            in_sems.at[c],
        )
        ins[c].start()

    def start_out(c):
        ins[c].wait()
        if c == 0:
            cy.wait()
            r = jax.lax.broadcasted_iota(jnp.int32, (8, _COLS), 0)
            cc = jax.lax.broadcasted_iota(jnp.int32, (8, _COLS), 1)
            buf[0:8, :] = jnp.where((r == 0) & (cc == 3), yv[0, 2], buf[0:8, :])
        outs[c] = pltpu.make_async_copy(
            buf.at[pl.ds((c % _K) * _CB, _CB)],
            o_hbm.at[pl.ds(c * _CB, _CB)],
            out_sems.at[c],
        )
        outs[c].start()

    for c in range(_NC):
        if c >= _K:
            outs[c - _K].wait()
        start_in(c)
        if c >= _L:
            start_out(c - _L)
    for c in range(_NC - _L, _NC):
        start_out(c)
    for c in range(max(_NC - _K, 0), _NC):
        outs[c].wait()


def kernel(x, y):
    return pl.pallas_call(
        _dma_pipeline,
        in_specs=[
            pl.BlockSpec(memory_space=pl.ANY),
            pl.BlockSpec(memory_space=pl.ANY),
        ],
        out_specs=pl.BlockSpec(memory_space=pl.ANY),
        out_shape=jax.ShapeDtypeStruct((_ROWS, _COLS), x.dtype),
        scratch_shapes=[
            pltpu.VMEM((_K * _CB, _COLS), jnp.float32),
            pltpu.VMEM((8, _COLS), jnp.float32),
            pltpu.SemaphoreType.DMA((_NC,)),
            pltpu.SemaphoreType.DMA((_NC,)),
            pltpu.SemaphoreType.DMA,
        ],
    )(x, y)


# speedup vs baseline: 49.8508x; 1.0077x over previous
import jax
import jax.numpy as jnp
from jax.experimental import pallas as pl
from jax.experimental.pallas import tpu as pltpu

_ROWS, _COLS = 16384, 1024
_CB = 1024
_NC = _ROWS // _CB
_K = 12
_L = 6


def _dma_pipeline(x_hbm, y_hbm, o_hbm, buf, yv, in_sems, out_sems, ysem):
    cy = pltpu.make_async_copy(y_hbm.at[pl.ds(0, 8)], yv, ysem)
    cy.start()

    ins = [None] * _NC
    outs = [None] * _NC

    def start_in(c):
        ins[c] = pltpu.make_async_copy(
            x_hbm.at[pl.ds(c * _CB, _CB)],
            buf.at[pl.ds((c % _K) * _CB, _CB)],
            in_sems.at[c],
        )
        ins[c].start()

    def start_out(c):
        ins[c].wait()
        if c == 0:
            cy.wait()
            r = jax.lax.broadcasted_iota(jnp.int32, (8, _COLS), 0)
            cc = jax.lax.broadcasted_iota(jnp.int32, (8, _COLS), 1)
            buf[0:8, :] = jnp.where((r == 0) & (cc == 3), yv[0, 2], buf[0:8, :])
        outs[c] = pltpu.make_async_copy(
            buf.at[pl.ds((c % _K) * _CB, _CB)],
            o_hbm.at[pl.ds(c * _CB, _CB)],
            out_sems.at[c],
        )
        outs[c].start()

    for c in range(_NC):
        if c >= _K:
            outs[c - _K].wait()
        start_in(c)
        if c >= _L:
            start_out(c - _L)
    for c in range(_NC - _L, _NC):
        start_out(c)
    for c in range(max(_NC - _K, 0), _NC):
        outs[c].wait()


def kernel(x, y):
    return pl.pallas_call(
        _dma_pipeline,
        in_specs=[
            pl.BlockSpec(memory_space=pl.ANY),
            pl.BlockSpec(memory_space=pl.ANY),
        ],
        out_specs=pl.BlockSpec(memory_space=pl.ANY),
        out_shape=jax.ShapeDtypeStruct((_ROWS, _COLS), x.dtype),
        scratch_shapes=[
            pltpu.VMEM((_K * _CB, _COLS), jnp.float32),
            pltpu.VMEM((8, _COLS), jnp.float32),
            pltpu.SemaphoreType.DMA((_NC,)),
            pltpu.SemaphoreType.DMA((_NC,)),
            pltpu.SemaphoreType.DMA,
        ],
    )(x, y)
